# baseline (speedup 1.0000x reference)
import functools

import jax
import jax.numpy as jnp
from jax import lax
from jax.experimental import pallas as pl
from jax.experimental.pallas import tpu as pltpu
from jax.experimental.pallas import tpu_sc as plsc

DIM_BATCH = 16
WIDTH_ENC = 4096
HALF = WIDTH_ENC // 2
LANES = 16


def _frames_body(x_hbm, r_hbm, lens_hbm, out_hbm, z_v, lens_v, out_v, sem):
    c = lax.axis_index("c")
    s = lax.axis_index("s")
    wid = s * 2 + c
    b = wid // 2
    h = wid % 2

    cp_l = pltpu.async_copy(lens_hbm, lens_v, sem)
    cp_x = pltpu.async_copy(x_hbm.at[b], z_v.at[pl.ds(0, WIDTH_ENC)], sem)
    cp_r = pltpu.async_copy(r_hbm.at[b], z_v.at[pl.ds(WIDTH_ENC, WIDTH_ENC)], sem)
    cp_l.wait()
    cp_x.wait()
    cp_r.wait()

    lane = lax.broadcasted_iota(jnp.int32, (LANES,), 0)
    base = plsc.load_gather(lens_v, [jnp.full((LANES,), b, jnp.int32)])
    base = base + h * HALF + lane

    @plsc.parallel_loop(0, HALF // LANES, unroll=1)
    def _(i):
        out_v[pl.ds(i * LANES, LANES)] = plsc.load_gather(z_v, [base + i * LANES])

    pltpu.sync_copy(out_v, out_hbm.at[b, pl.ds(h * HALF, HALF)])


@jax.jit
def _frames_sc(x, ragged_dense, lens):
    mesh = plsc.VectorSubcoreMesh(core_axis_name="c", subcore_axis_name="s")
    run = functools.partial(
        pl.kernel,
        mesh=mesh,
        out_type=jax.ShapeDtypeStruct((DIM_BATCH, WIDTH_ENC), jnp.float32),
        scratch_types=[
            pltpu.VMEM((2 * WIDTH_ENC,), jnp.float32),
            pltpu.VMEM((LANES,), jnp.int32),
            pltpu.VMEM((HALF,), jnp.float32),
            pltpu.SemaphoreType.DMA,
        ],
        compiler_params=pltpu.CompilerParams(
            needs_layout_passes=False,
            disable_bounds_checks=True,
        ),
    )(_frames_body)
    return run(x, ragged_dense, lens)


def kernel(x, ragged_dense, lens):
    y = _frames_sc(x, ragged_dense, lens)
    return y, lens[:, None]


# speedup vs baseline: 1.0163x; 1.0163x over previous
import functools

import jax
import jax.numpy as jnp
from jax import lax
from jax.experimental import pallas as pl
from jax.experimental.pallas import tpu as pltpu
from jax.experimental.pallas import tpu_sc as plsc

DIM_BATCH = 16
WIDTH_ENC = 4096
HALF = WIDTH_ENC // 2
LANES = 16


def _frames_body(x_hbm, r_hbm, lens_hbm, out_hbm, z_v, lens_v, out_v, sem):
    c = lax.axis_index("c")
    s = lax.axis_index("s")
    wid = s * 2 + c
    b = wid // 2
    h = wid % 2

    cp_l = pltpu.async_copy(lens_hbm, lens_v, sem)
    cp_x = pltpu.async_copy(x_hbm.at[b], z_v.at[pl.ds(0, WIDTH_ENC)], sem)
    cp_r = pltpu.async_copy(r_hbm.at[b], z_v.at[pl.ds(WIDTH_ENC, WIDTH_ENC)], sem)
    cp_l.wait()
    cp_x.wait()
    cp_r.wait()

    lane = lax.broadcasted_iota(jnp.int32, (LANES,), 0)
    base = plsc.load_gather(lens_v, [jnp.full((LANES,), b, jnp.int32)])
    base = base + h * HALF + lane

    @plsc.parallel_loop(0, HALF // LANES, unroll=8)
    def _(i):
        out_v[pl.ds(i * LANES, LANES)] = plsc.load_gather(z_v, [base + i * LANES])

    pltpu.sync_copy(out_v, out_hbm.at[b, pl.ds(h * HALF, HALF)])


@jax.jit
def _frames_sc(x, ragged_dense, lens):
    mesh = plsc.VectorSubcoreMesh(core_axis_name="c", subcore_axis_name="s")
    run = functools.partial(
        pl.kernel,
        mesh=mesh,
        out_type=jax.ShapeDtypeStruct((DIM_BATCH, WIDTH_ENC), jnp.float32),
        scratch_types=[
            pltpu.VMEM((2 * WIDTH_ENC,), jnp.float32),
            pltpu.VMEM((LANES,), jnp.int32),
            pltpu.VMEM((HALF,), jnp.float32),
            pltpu.SemaphoreType.DMA,
        ],
        compiler_params=pltpu.CompilerParams(
            needs_layout_passes=False,
            disable_bounds_checks=True,
            skip_device_barrier=True,
        ),
    )(_frames_body)
    return run(x, ragged_dense, lens)


def kernel(x, ragged_dense, lens):
    y = _frames_sc(x, ragged_dense, lens)
    return y, lens[:, None]


# speedup vs baseline: 1.1033x; 1.0856x over previous
import functools

import jax
import jax.numpy as jnp
from jax import lax
from jax.experimental import pallas as pl
from jax.experimental.pallas import tpu as pltpu
from jax.experimental.pallas import tpu_sc as plsc

DIM_BATCH = 16
WIDTH_ENC = 4096
LANES = 16


def _frames_body(x_hbm, r_hbm, lens_hbm, out_hbm, z_v, lens_v, out_v, sem):
    b = lax.axis_index("s")

    cp_l = pltpu.async_copy(lens_hbm, lens_v, sem)
    cp_x = pltpu.async_copy(x_hbm.at[b], z_v.at[pl.ds(0, WIDTH_ENC)], sem)
    cp_r = pltpu.async_copy(r_hbm.at[b], z_v.at[pl.ds(WIDTH_ENC, WIDTH_ENC)], sem)
    cp_l.wait()
    cp_x.wait()
    cp_r.wait()

    lane = lax.broadcasted_iota(jnp.int32, (LANES,), 0)
    base = plsc.load_gather(lens_v, [jnp.full((LANES,), b, jnp.int32)])
    base = base + lane

    @plsc.parallel_loop(0, WIDTH_ENC // LANES, unroll=8)
    def _(i):
        out_v[pl.ds(i * LANES, LANES)] = plsc.load_gather(z_v, [base + i * LANES])

    pltpu.sync_copy(out_v, out_hbm.at[b])


@jax.jit
def _frames_sc(x, ragged_dense, lens):
    mesh = plsc.VectorSubcoreMesh(
        core_axis_name="c", subcore_axis_name="s", num_cores=1
    )
    run = functools.partial(
        pl.kernel,
        mesh=mesh,
        out_type=jax.ShapeDtypeStruct((DIM_BATCH, WIDTH_ENC), jnp.float32),
        scratch_types=[
            pltpu.VMEM((2 * WIDTH_ENC,), jnp.float32),
            pltpu.VMEM((LANES,), jnp.int32),
            pltpu.VMEM((WIDTH_ENC,), jnp.float32),
            pltpu.SemaphoreType.DMA,
        ],
        compiler_params=pltpu.CompilerParams(
            needs_layout_passes=False,
            disable_bounds_checks=True,
            skip_device_barrier=True,
        ),
    )(_frames_body)
    return run(x, ragged_dense, lens)


def kernel(x, ragged_dense, lens):
    y = _frames_sc(x, ragged_dense, lens)
    return y, lens[:, None]


# speedup vs baseline: 1.1629x; 1.0540x over previous
import functools

import jax
import jax.numpy as jnp
from jax import lax
from jax.experimental import pallas as pl
from jax.experimental.pallas import tpu as pltpu
from jax.experimental.pallas import tpu_sc as plsc

DIM_BATCH = 16
WIDTH_ENC = 4096
LANES = 16


def _frames_body(x_hbm, r_hbm, lens_hbm, out_hbm, out2_hbm, z_v, lens_v, out_v, sem):
    b = lax.axis_index("s")

    cp_l = pltpu.async_copy(lens_hbm, lens_v, sem)
    cp_x = pltpu.async_copy(x_hbm.at[b], z_v.at[pl.ds(0, WIDTH_ENC)], sem)
    cp_r = pltpu.async_copy(r_hbm.at[b], z_v.at[pl.ds(WIDTH_ENC, WIDTH_ENC)], sem)
    cp_l.wait()
    cp_x.wait()
    cp_r.wait()

    lane = lax.broadcasted_iota(jnp.int32, (LANES,), 0)
    base = plsc.load_gather(lens_v, [jnp.full((LANES,), b, jnp.int32)])
    base = base + lane

    @plsc.parallel_loop(0, WIDTH_ENC // LANES, unroll=8)
    def _(i):
        out_v[pl.ds(i * LANES, LANES)] = plsc.load_gather(z_v, [base + i * LANES])

    pltpu.sync_copy(out_v, out_hbm.at[b])

    @pl.when(b == 0)
    def _():
        pltpu.sync_copy(lens_v, out2_hbm.at[0])


@jax.jit
def _frames_sc(x, ragged_dense, lens):
    mesh = plsc.VectorSubcoreMesh(
        core_axis_name="c", subcore_axis_name="s", num_cores=1
    )
    run = functools.partial(
        pl.kernel,
        mesh=mesh,
        out_type=(
            jax.ShapeDtypeStruct((DIM_BATCH, WIDTH_ENC), jnp.float32),
            jax.ShapeDtypeStruct((1, DIM_BATCH), jnp.int32),
        ),
        scratch_types=[
            pltpu.VMEM((2 * WIDTH_ENC,), jnp.float32),
            pltpu.VMEM((LANES,), jnp.int32),
            pltpu.VMEM((WIDTH_ENC,), jnp.float32),
            pltpu.SemaphoreType.DMA,
        ],
        compiler_params=pltpu.CompilerParams(
            needs_layout_passes=False,
            disable_bounds_checks=True,
            skip_device_barrier=True,
        ),
    )(_frames_body)
    return run(x, ragged_dense, lens)


def kernel(x, ragged_dense, lens):
    y, l2 = _frames_sc(x, ragged_dense, lens)
    return y, l2.reshape(DIM_BATCH, 1)
